# baseline (speedup 1.0000x reference)
import functools

import jax
import jax.numpy as jnp
from jax import lax
from jax.experimental import pallas as pl
from jax.experimental.pallas import tpu as pltpu
from jax.experimental.pallas import tpu_sc as plsc

_K = 8192
_D = 32
_N = 8192
_TILE = 256
_BETA = 0.25

_NC = 2
_NS = 16
_NW = _NC * _NS
_BPW = _N // _NW
_GCH = 128
_CPW = _BPW // _GCH
_DPAD = 128


def _dist_argmin_body(z2_ref, e2_ref, kiota_ref, z_ref, emb_ref, idx_ref,
                      dsum_ref):
    mm2 = lax.dot_general(z_ref[...], emb_ref[...],
                          dimension_numbers=(((1,), (1,)), ((), ())))
    d = (z2_ref[...] + e2_ref[...]) - mm2
    dmin = jnp.min(d, axis=1, keepdims=True)
    idxf = jnp.min(jnp.where(d == dmin, kiota_ref[...], float(_K)), axis=1)
    idx = idxf.astype(jnp.int32)
    i = pl.program_id(0)
    idx_ref[i // (1024 // _TILE), pl.ds((i % (1024 // _TILE)) * _TILE, _TILE)] = idx

    @pl.when(i == 0)
    def _init():
        dsum_ref[0, 0] = 0.0

    dsum_ref[0, 0] += jnp.sum(dmin)


_dist_call = pl.pallas_call(
    _dist_argmin_body,
    grid=(_N // _TILE,),
    in_specs=[
        pl.BlockSpec((_TILE, 1), lambda i: (i, 0)),
        pl.BlockSpec((1, _K), lambda i: (0, 0)),
        pl.BlockSpec((1, _K), lambda i: (0, 0)),
        pl.BlockSpec((_TILE, _D), lambda i: (i, 0)),
        pl.BlockSpec((_K, _D), lambda i: (0, 0)),
    ],
    out_specs=[
        pl.BlockSpec((8, 1024), lambda i: (0, 0)),
        pl.BlockSpec((1, 1), lambda i: (0, 0), memory_space=pltpu.SMEM),
    ],
    out_shape=[
        jax.ShapeDtypeStruct((8, 1024), jnp.int32),
        jax.ShapeDtypeStruct((1, 1), jnp.float32),
    ],
)


@functools.cache
def _build_gather_kernel():
    @functools.partial(
        pl.kernel,
        mesh=plsc.VectorSubcoreMesh(core_axis_name="c", subcore_axis_name="s"),
        out_type=jax.ShapeDtypeStruct((_N, _DPAD), jnp.float32),
        scratch_types=[
            pltpu.VMEM((_CPW, _GCH), jnp.int32),
            pltpu.VMEM((_BPW, _DPAD), jnp.float32),
            pltpu.SemaphoreType.DMA,
        ],
    )
    def _gather_kernel(idx_hbm, table_hbm, out_hbm, idx_v, rows_v, sem):
        wid = lax.axis_index("s") * _NC + lax.axis_index("c")
        row = wid // 4
        col = (wid % 4) * _BPW
        for j in range(_CPW):
            pltpu.sync_copy(idx_hbm.at[row, pl.ds(col + j * _GCH, _GCH)],
                            idx_v.at[j])
        copies = [
            pltpu.async_copy(table_hbm.at[idx_v.at[j]],
                             rows_v.at[pl.ds(j * _GCH, _GCH)], sem)
            for j in range(_CPW)
        ]
        for c in copies:
            c.wait()
        pltpu.sync_copy(rows_v, out_hbm.at[pl.ds(wid * _BPW, _BPW)])

    return _gather_kernel


def _st_transpose_body(z_ref, q_ref, out_ref):
    q = q_ref[:, 0:_D]
    st = z_ref[...] + (q - z_ref[...])
    out_ref[0] = st.T


_st_call = pl.pallas_call(
    _st_transpose_body,
    grid=(8,),
    in_specs=[
        pl.BlockSpec((_N // 8, _D), lambda i: (i, 0)),
        pl.BlockSpec((_N // 8, _DPAD), lambda i: (i, 0)),
    ],
    out_specs=pl.BlockSpec((1, _D, _N // 8), lambda i: (i, 0, 0)),
    out_shape=jax.ShapeDtypeStruct((8, _D, _N // 8), jnp.float32),
)


def kernel(z, embedding):
    b, c, h, w = z.shape
    zf = jnp.transpose(z, (0, 2, 3, 1)).reshape(-1, _D)
    z2 = jnp.sum(zf ** 2, axis=1, keepdims=True)
    e2 = jnp.sum(embedding ** 2, axis=1).reshape(1, _K)
    kiota = lax.broadcasted_iota(jnp.float32, (1, _K), 1)

    idx_out, dsum = _dist_call(z2, e2, kiota, zf + zf, embedding)

    emb_pad = jnp.pad(embedding, ((0, 0), (0, _DPAD - _D)))
    q = _build_gather_kernel()(idx_out, emb_pad)

    out = _st_call(zf, q).reshape(b, c, h, w)

    mean_min_dist = dsum[0, 0] / (_N * _D)
    loss = mean_min_dist + _BETA * mean_min_dist
    return (out, loss, idx_out)


# speedup vs baseline: 1.0962x; 1.0962x over previous
import functools

import jax
import jax.numpy as jnp
from jax import lax
from jax.experimental import pallas as pl
from jax.experimental.pallas import tpu as pltpu
from jax.experimental.pallas import tpu_sc as plsc

_K = 8192
_D = 32
_N = 8192
_TILE = 512
_BETA = 0.25

_NC = 2
_NS = 16
_NW = _NC * _NS
_BPW = _N // _NW
_GCH = 128
_CPW = _BPW // _GCH
_DPAD = 128


def _dist_argmin_body(z2_ref, e2_ref, z_ref, emb_ref, idx_ref, dsum_ref):
    mm = lax.dot_general(z_ref[...], emb_ref[...],
                         dimension_numbers=(((1,), (1,)), ((), ())))
    d = (z2_ref[...] + e2_ref[...]) - 2.0 * mm
    dmin = jnp.min(d, axis=1, keepdims=True)
    kiota = lax.broadcasted_iota(jnp.int32, (_TILE, _K), 1)
    idx = jnp.min(jnp.where(d == dmin, kiota, _K), axis=1)
    i = pl.program_id(0)
    idx_ref[i // (1024 // _TILE), pl.ds((i % (1024 // _TILE)) * _TILE, _TILE)] = idx

    @pl.when(i == 0)
    def _init():
        dsum_ref[0, 0] = 0.0

    dsum_ref[0, 0] += jnp.sum(dmin)


_dist_call = pl.pallas_call(
    _dist_argmin_body,
    grid=(_N // _TILE,),
    in_specs=[
        pl.BlockSpec((_TILE, 1), lambda i: (i, 0)),
        pl.BlockSpec((1, _K), lambda i: (0, 0)),
        pl.BlockSpec((_TILE, _D), lambda i: (i, 0)),
        pl.BlockSpec((_K, _D), lambda i: (0, 0)),
    ],
    out_specs=[
        pl.BlockSpec((8, 1024), lambda i: (0, 0)),
        pl.BlockSpec((1, 1), lambda i: (0, 0), memory_space=pltpu.SMEM),
    ],
    out_shape=[
        jax.ShapeDtypeStruct((8, 1024), jnp.int32),
        jax.ShapeDtypeStruct((1, 1), jnp.float32),
    ],
)


@functools.cache
def _build_gather_kernel():
    @functools.partial(
        pl.kernel,
        mesh=plsc.VectorSubcoreMesh(core_axis_name="c", subcore_axis_name="s"),
        out_type=jax.ShapeDtypeStruct((_N, _DPAD), jnp.float32),
        scratch_types=[
            pltpu.VMEM((_CPW, _GCH), jnp.int32),
            pltpu.VMEM((_BPW, _DPAD), jnp.float32),
            pltpu.SemaphoreType.DMA,
        ],
    )
    def _gather_kernel(idx_hbm, table_hbm, out_hbm, idx_v, rows_v, sem):
        wid = lax.axis_index("s") * _NC + lax.axis_index("c")
        row = wid // 4
        col = (wid % 4) * _BPW
        for j in range(_CPW):
            pltpu.sync_copy(idx_hbm.at[row, pl.ds(col + j * _GCH, _GCH)],
                            idx_v.at[j])
        copies = [
            pltpu.async_copy(table_hbm.at[idx_v.at[j]],
                             rows_v.at[pl.ds(j * _GCH, _GCH)], sem)
            for j in range(_CPW)
        ]
        for c in copies:
            c.wait()
        pltpu.sync_copy(rows_v, out_hbm.at[pl.ds(wid * _BPW, _BPW)])

    return _gather_kernel


def _st_transpose_body(z_ref, q_ref, out_ref):
    q = q_ref[:, 0:_D]
    st = z_ref[...] + (q - z_ref[...])
    out_ref[0] = st.T


_st_call = pl.pallas_call(
    _st_transpose_body,
    grid=(8,),
    in_specs=[
        pl.BlockSpec((_N // 8, _D), lambda i: (i, 0)),
        pl.BlockSpec((_N // 8, _DPAD), lambda i: (i, 0)),
    ],
    out_specs=pl.BlockSpec((1, _D, _N // 8), lambda i: (i, 0, 0)),
    out_shape=jax.ShapeDtypeStruct((8, _D, _N // 8), jnp.float32),
)


def kernel(z, embedding):
    b, c, h, w = z.shape
    zf = jnp.transpose(z, (0, 2, 3, 1)).reshape(-1, _D)
    z2 = jnp.sum(zf ** 2, axis=1, keepdims=True)
    e2 = jnp.sum(embedding ** 2, axis=1).reshape(1, _K)
    idx_out, dsum = _dist_call(z2, e2, zf, embedding)

    emb_pad = jnp.pad(embedding, ((0, 0), (0, _DPAD - _D)))
    q = _build_gather_kernel()(idx_out, emb_pad)

    out = _st_call(zf, q).reshape(b, c, h, w)

    mean_min_dist = dsum[0, 0] / (_N * _D)
    loss = mean_min_dist + _BETA * mean_min_dist
    return (out, loss, idx_out)


# speedup vs baseline: 1.1132x; 1.0156x over previous
import functools

import jax
import jax.numpy as jnp
from jax import lax
from jax.experimental import pallas as pl
from jax.experimental.pallas import tpu as pltpu
from jax.experimental.pallas import tpu_sc as plsc

_K = 8192
_D = 32
_N = 8192
_TILE = 1024
_BETA = 0.25

_NC = 2
_NS = 16
_NW = _NC * _NS
_BPW = _N // _NW
_GCH = 128
_CPW = _BPW // _GCH
_DPAD = 128


def _dist_argmin_body(z2_ref, e2_ref, z_ref, emb_ref, idx_ref, dsum_ref):
    mm = lax.dot_general(z_ref[...], emb_ref[...],
                         dimension_numbers=(((1,), (1,)), ((), ())))
    d = (z2_ref[...] + e2_ref[...]) - 2.0 * mm
    dmin = jnp.min(d, axis=1, keepdims=True)
    kiota = lax.broadcasted_iota(jnp.int32, (_TILE, _K), 1)
    idx = jnp.min(jnp.where(d == dmin, kiota, _K), axis=1)
    i = pl.program_id(0)
    idx_ref[i // (1024 // _TILE), pl.ds((i % (1024 // _TILE)) * _TILE, _TILE)] = idx

    @pl.when(i == 0)
    def _init():
        dsum_ref[0, 0] = 0.0

    dsum_ref[0, 0] += jnp.sum(dmin)


_dist_call = pl.pallas_call(
    _dist_argmin_body,
    grid=(_N // _TILE,),
    in_specs=[
        pl.BlockSpec((_TILE, 1), lambda i: (i, 0)),
        pl.BlockSpec((1, _K), lambda i: (0, 0)),
        pl.BlockSpec((_TILE, _D), lambda i: (i, 0)),
        pl.BlockSpec((_K, _D), lambda i: (0, 0)),
    ],
    out_specs=[
        pl.BlockSpec((8, 1024), lambda i: (0, 0)),
        pl.BlockSpec((1, 1), lambda i: (0, 0), memory_space=pltpu.SMEM),
    ],
    out_shape=[
        jax.ShapeDtypeStruct((8, 1024), jnp.int32),
        jax.ShapeDtypeStruct((1, 1), jnp.float32),
    ],
)


@functools.cache
def _build_gather_kernel():
    @functools.partial(
        pl.kernel,
        mesh=plsc.VectorSubcoreMesh(core_axis_name="c", subcore_axis_name="s"),
        out_type=jax.ShapeDtypeStruct((_N, _DPAD), jnp.float32),
        scratch_types=[
            pltpu.VMEM((_CPW, _GCH), jnp.int32),
            pltpu.VMEM((_BPW, _DPAD), jnp.float32),
            pltpu.SemaphoreType.DMA,
        ],
    )
    def _gather_kernel(idx_hbm, table_hbm, out_hbm, idx_v, rows_v, sem):
        wid = lax.axis_index("s") * _NC + lax.axis_index("c")
        row = wid // 4
        col = (wid % 4) * _BPW
        for j in range(_CPW):
            pltpu.sync_copy(idx_hbm.at[row, pl.ds(col + j * _GCH, _GCH)],
                            idx_v.at[j])
        copies = [
            pltpu.async_copy(table_hbm.at[idx_v.at[j]],
                             rows_v.at[pl.ds(j * _GCH, _GCH)], sem)
            for j in range(_CPW)
        ]
        for c in copies:
            c.wait()
        pltpu.sync_copy(rows_v, out_hbm.at[pl.ds(wid * _BPW, _BPW)])

    return _gather_kernel


def _st_transpose_body(z_ref, q_ref, out_ref):
    q = q_ref[:, 0:_D]
    st = z_ref[...] + (q - z_ref[...])
    out_ref[0] = st.T


_st_call = pl.pallas_call(
    _st_transpose_body,
    grid=(8,),
    in_specs=[
        pl.BlockSpec((_N // 8, _D), lambda i: (i, 0)),
        pl.BlockSpec((_N // 8, _DPAD), lambda i: (i, 0)),
    ],
    out_specs=pl.BlockSpec((1, _D, _N // 8), lambda i: (i, 0, 0)),
    out_shape=jax.ShapeDtypeStruct((8, _D, _N // 8), jnp.float32),
)


def kernel(z, embedding):
    b, c, h, w = z.shape
    zf = jnp.transpose(z, (0, 2, 3, 1)).reshape(-1, _D)
    z2 = jnp.sum(zf ** 2, axis=1, keepdims=True)
    e2 = jnp.sum(embedding ** 2, axis=1).reshape(1, _K)
    idx_out, dsum = _dist_call(z2, e2, zf, embedding)

    emb_pad = jnp.pad(embedding, ((0, 0), (0, _DPAD - _D)))
    q = _build_gather_kernel()(idx_out, emb_pad)

    out = _st_call(zf, q).reshape(b, c, h, w)

    mean_min_dist = dsum[0, 0] / (_N * _D)
    loss = mean_min_dist + _BETA * mean_min_dist
    return (out, loss, idx_out)


# speedup vs baseline: 1.1293x; 1.0145x over previous
import functools

import jax
import jax.numpy as jnp
from jax import lax
from jax.experimental import pallas as pl
from jax.experimental.pallas import tpu as pltpu
from jax.experimental.pallas import tpu_sc as plsc

_K = 8192
_D = 32
_N = 8192
_TILE = 1024
_BETA = 0.25

_NC = 2
_NS = 16
_NW = _NC * _NS
_BPW = _N // _NW
_GCH = 128
_CPW = _BPW // _GCH
_DPAD = 128


def _dist_argmin_body(z2_ref, e2_ref, z_ref, emb_ref, idx_ref, dsum_ref):
    mm = lax.dot_general(z_ref[...], emb_ref[...],
                         dimension_numbers=(((1,), (1,)), ((), ())))
    d = (z2_ref[...] + e2_ref[...]) - 2.0 * mm
    dmin = jnp.min(d, axis=1, keepdims=True)
    kiota = lax.broadcasted_iota(jnp.int32, (_TILE, _K), 1)
    idx = jnp.min(jnp.where(d == dmin, kiota, _K), axis=1)
    i = pl.program_id(0)
    idx_ref[i // (1024 // _TILE), pl.ds((i % (1024 // _TILE)) * _TILE, _TILE)] = idx

    @pl.when(i == 0)
    def _init():
        dsum_ref[0, 0] = 0.0

    dsum_ref[0, 0] += jnp.sum(dmin)


_dist_call = pl.pallas_call(
    _dist_argmin_body,
    grid=(_N // _TILE,),
    in_specs=[
        pl.BlockSpec((_TILE, 1), lambda i: (i, 0)),
        pl.BlockSpec((1, _K), lambda i: (0, 0)),
        pl.BlockSpec((_TILE, _D), lambda i: (i, 0)),
        pl.BlockSpec((_K, _D), lambda i: (0, 0)),
    ],
    out_specs=[
        pl.BlockSpec((8, 1024), lambda i: (0, 0)),
        pl.BlockSpec((1, 1), lambda i: (0, 0), memory_space=pltpu.SMEM),
    ],
    out_shape=[
        jax.ShapeDtypeStruct((8, 1024), jnp.int32),
        jax.ShapeDtypeStruct((1, 1), jnp.float32),
    ],
)


@functools.cache
def _build_gather_kernel():
    @functools.partial(
        pl.kernel,
        mesh=plsc.VectorSubcoreMesh(core_axis_name="c", subcore_axis_name="s"),
        out_type=jax.ShapeDtypeStruct((_N, _DPAD), jnp.float32),
        scratch_types=[
            pltpu.VMEM((_CPW, _GCH), jnp.int32),
            pltpu.VMEM((_BPW, _DPAD), jnp.float32),
            pltpu.SemaphoreType.DMA,
        ],
    )
    def _gather_kernel(idx_hbm, table_hbm, out_hbm, idx_v, rows_v, sem):
        wid = lax.axis_index("s") * _NC + lax.axis_index("c")
        row = wid // 4
        col = (wid % 4) * _BPW
        for j in range(_CPW):
            pltpu.sync_copy(idx_hbm.at[row, pl.ds(col + j * _GCH, _GCH)],
                            idx_v.at[j])
        copies = [
            pltpu.async_copy(table_hbm.at[idx_v.at[j]],
                             rows_v.at[pl.ds(j * _GCH, _GCH)], sem)
            for j in range(_CPW)
        ]
        for c in copies:
            c.wait()
        pltpu.sync_copy(rows_v, out_hbm.at[pl.ds(wid * _BPW, _BPW)])

    return _gather_kernel


def _st_transpose_body(z_ref, q_ref, out_ref):
    q = q_ref[:, 0:_D]
    st = z_ref[...] + (q - z_ref[...])
    out_ref[0] = st.T.reshape(_D, 32, 32)


_st_call = pl.pallas_call(
    _st_transpose_body,
    grid=(8,),
    in_specs=[
        pl.BlockSpec((_N // 8, _D), lambda i: (i, 0)),
        pl.BlockSpec((_N // 8, _DPAD), lambda i: (i, 0)),
    ],
    out_specs=pl.BlockSpec((1, _D, 32, 32), lambda i: (i, 0, 0, 0)),
    out_shape=jax.ShapeDtypeStruct((8, _D, 32, 32), jnp.float32),
)


def kernel(z, embedding):
    b, c, h, w = z.shape
    zf = jnp.transpose(z, (0, 2, 3, 1)).reshape(-1, _D)
    z2 = jnp.sum(zf ** 2, axis=1, keepdims=True)
    e2 = jnp.sum(embedding ** 2, axis=1).reshape(1, _K)
    idx_out, dsum = _dist_call(z2, e2, zf, embedding)

    emb_pad = jnp.pad(embedding, ((0, 0), (0, _DPAD - _D)))
    q = _build_gather_kernel()(idx_out, emb_pad)

    out = _st_call(zf, q)

    mean_min_dist = dsum[0, 0] / (_N * _D)
    loss = mean_min_dist + _BETA * mean_min_dist
    return (out, loss, idx_out)
